# baseline (speedup 1.0000x reference)
import functools

import jax
import jax.numpy as jnp
from jax import lax
from jax.experimental import pallas as pl
from jax.experimental.pallas import tpu as pltpu
from jax.experimental.pallas import tpu_sc as plsc

B, S = 32, 200
NQ, NS = 100000, 2000
NBQ, NBS = 8, 8
D = 64
SB = B * S
NW = 32
CHUNK = 128
TCHUNK = 100

NS_PAD = 2048
SK_PER_W = NS_PAD // NW
SK_ROWS = SK_PER_W * NBS
SK_CHUNKS = SK_ROWS // CHUNK

TOK_PER_W = SB // NW
TOK_CHUNKS = TOK_PER_W // TCHUNK

N1_CHUNKS = TOK_PER_W * NBQ // TCHUNK
N_SUPER = 4
CH_PER_SUPER = N1_CHUNKS // N_SUPER
SUPER_ROWS = CH_PER_SUPER * TCHUNK
TOK_PER_SUPER = SUPER_ROWS // NBQ


def _wid():
    return lax.axis_index("s") * 2 + lax.axis_index("c")


@functools.cache
def _sc_kernels():
    mesh = plsc.VectorSubcoreMesh(core_axis_name="c", subcore_axis_name="s")
    return _make_k1(mesh), _make_k3(mesh)


def _make_k1(mesh):
  return functools.partial(
    pl.kernel,
    out_type=(
        jax.ShapeDtypeStruct((NS_PAD, D), jnp.float32),
        jax.ShapeDtypeStruct((SB, NBQ), jnp.int32),
    ),
    mesh=mesh,
    compiler_params=pltpu.CompilerParams(use_tc_tiling_on_sc=False),
    scratch_types=[
        pltpu.VMEM((SK_CHUNKS, CHUNK), jnp.int32),
        pltpu.VMEM((TOK_CHUNKS, TCHUNK), jnp.int32),
        pltpu.VMEM((SK_ROWS, D), jnp.float32),
        pltpu.VMEM((SK_PER_W, D), jnp.float32),
        pltpu.VMEM((TOK_PER_W, NBQ), jnp.int32),
        pltpu.SemaphoreType.DMA,
        pltpu.SemaphoreType.DMA,
    ],
  )(_k1_body)


def _k1_body(emb_q_hbm, qnbrs_hbm, idx_s_hbm, idx_q_hbm,
             m2_out, qnb_out,
             idxs_v, idxq_v, rows_v, acc_v, qnb_v, sem0, sem1):
    wid = _wid()

    pltpu.sync_copy(idx_s_hbm.at[pl.ds(wid * SK_CHUNKS, SK_CHUNKS)], idxs_v)
    pltpu.sync_copy(idx_q_hbm.at[pl.ds(wid * TOK_CHUNKS, TOK_CHUNKS)], idxq_v)
    m2_cps = [
        pltpu.async_copy(emb_q_hbm.at[idxs_v.at[ch]],
                         rows_v.at[pl.ds(ch * CHUNK, CHUNK)], sem0)
        for ch in range(SK_CHUNKS)
    ]
    tok_cps = [
        pltpu.async_copy(qnbrs_hbm.at[idxq_v.at[ch]],
                         qnb_v.at[pl.ds(ch * TCHUNK, TCHUNK)], sem1)
        for ch in range(TOK_CHUNKS)
    ]
    for cp in m2_cps:
        cp.wait()

    def _acc_skill(sk, _):
        for c4 in range(D // 16):
            col = pl.ds(c4 * 16, 16)
            v = rows_v[sk * NBS, col]
            for j in range(1, NBS):
                v = v + rows_v[sk * NBS + j, col]
            acc_v[sk, col] = v * (1.0 / NBS)
        return 0

    lax.fori_loop(0, SK_PER_W, _acc_skill, 0)
    pltpu.sync_copy(acc_v, m2_out.at[pl.ds(wid * SK_PER_W, SK_PER_W)])

    for cp in tok_cps:
        cp.wait()
    pltpu.sync_copy(qnb_v, qnb_out.at[pl.ds(wid * TOK_PER_W, TOK_PER_W)])


def _make_k3(mesh):
  return functools.partial(
    pl.kernel,
    out_type=(
        jax.ShapeDtypeStruct((SB, D), jnp.float32),
        jax.ShapeDtypeStruct((SB, D), jnp.float32),
    ),
    mesh=mesh,
    compiler_params=pltpu.CompilerParams(use_tc_tiling_on_sc=False),
    scratch_types=[
        pltpu.VMEM((N1_CHUNKS, TCHUNK), jnp.int32),
        pltpu.VMEM((TOK_CHUNKS, TCHUNK), jnp.int32),
        pltpu.VMEM((SUPER_ROWS, D), jnp.float32),
        pltpu.VMEM((SUPER_ROWS, D), jnp.float32),
        pltpu.VMEM((TOK_PER_W, D), jnp.float32),
        pltpu.VMEM((TOK_PER_W, D), jnp.float32),
        pltpu.VMEM_SHARED((NS, D), jnp.float32),
        pltpu.SemaphoreType.DMA,
        pltpu.SemaphoreType.DMA,
        pltpu.SemaphoreType.DMA,
    ],
  )(_k3_body)


def _k3_body(a1_hbm, idx3_hbm, emb_q_hbm, idx_q_hbm,
             n1_out, e0_out,
             idx_v, idxq_v, rows0_v, rows1_v, acc_v, e0_v,
             a1_sp, sem0, sem1, sem2):
    wid = _wid()
    sid = lax.axis_index("s")

    pltpu.sync_copy(idx_q_hbm.at[pl.ds(wid * TOK_CHUNKS, TOK_CHUNKS)], idxq_v)
    e0_cps = [
        pltpu.async_copy(emb_q_hbm.at[idxq_v.at[ch]],
                         e0_v.at[pl.ds(ch * TCHUNK, TCHUNK)], sem2)
        for ch in range(TOK_CHUNKS)
    ]

    @pl.when(sid == 0)
    def _():
        pltpu.sync_copy(a1_hbm, a1_sp)

    pltpu.sync_copy(idx3_hbm.at[pl.ds(wid * N1_CHUNKS, N1_CHUNKS)], idx_v)
    plsc.subcore_barrier()
    bufs = [rows0_v, rows1_v]
    sems = [sem0, sem1]

    def fire(sc):
        buf, sem = bufs[sc % 2], sems[sc % 2]
        return [
            pltpu.async_copy(
                a1_sp.at[idx_v.at[sc * CH_PER_SUPER + ch]],
                buf.at[pl.ds(ch * TCHUNK, TCHUNK)], sem)
            for ch in range(CH_PER_SUPER)
        ]

    pending = fire(0)
    for sc in range(N_SUPER):
        for cp in pending:
            cp.wait()
        if sc + 1 < N_SUPER:
            pending = fire(sc + 1)
        buf = bufs[sc % 2]

        def _acc_tok(tl, _):
            for c4 in range(D // 16):
                col = pl.ds(c4 * 16, 16)
                v = buf[tl * NBQ, col]
                for j in range(1, NBQ):
                    v = v + buf[tl * NBQ + j, col]
                acc_v[sc * TOK_PER_SUPER + tl, col] = v * (1.0 / NBQ)
            return 0

        lax.fori_loop(0, TOK_PER_SUPER, _acc_tok, 0)

    pltpu.sync_copy(acc_v, n1_out.at[pl.ds(wid * TOK_PER_W, TOK_PER_W)])
    for cp in e0_cps:
        cp.wait()
    pltpu.sync_copy(e0_v, e0_out.at[pl.ds(wid * TOK_PER_W, TOK_PER_W)])


def _k2_body(m2_ref, emb_s_ref, w0_ref, b0_ref, out_ref):
    u = emb_s_ref[...] + m2_ref[:NS, :]
    out_ref[...] = jax.nn.relu(
        jnp.dot(u, w0_ref[...], preferred_element_type=jnp.float32)
        + b0_ref[...])


def _k2(m2, emb_s, w0, b0):
    return pl.pallas_call(
        _k2_body,
        out_shape=jax.ShapeDtypeStruct((NS, D), jnp.float32),
    )(m2, emb_s, w0, b0)


def _k4_body(e0_ref, n1_ref, resp_ref, w1_ref, b1_ref, wi_ref, wh_ref,
             bl_ref, embr_ref, out_ref, xw_s, qa_s, hs_s):
    e0 = e0_ref[:SB, :]
    n1 = n1_ref[:SB, :]
    q_agg = jax.nn.relu(
        jnp.dot(e0 + n1, w1_ref[...], preferred_element_type=jnp.float32)
        + b1_ref[...])
    qa_s[...] = q_agg

    wi_top = wi_ref[:D, :]
    wi_bot = wi_ref[D:, :]
    v0 = jnp.dot(embr_ref[0:1, :], wi_bot,
                 preferred_element_type=jnp.float32) + bl_ref[...]
    v1 = jnp.dot(embr_ref[1:2, :], wi_bot,
                 preferred_element_type=jnp.float32) + bl_ref[...]
    xw_s[...] = (jnp.dot(q_agg, wi_top, preferred_element_type=jnp.float32)
                 + v0 + resp_ref[...] * (v1 - v0))

    wh = wh_ref[...]

    UNROLL = 4
    HB = B // 2

    def step4(t4, carry):
        h1, c1, h2, c2 = carry
        for k in range(UNROLL):
            t = t4 * UNROLL + k

            def half(h, c, off):
                xt = xw_s[pl.ds(t * B + off, HB), :]
                gates = xt + jnp.dot(h, wh,
                                     preferred_element_type=jnp.float32)
                i_f = jax.nn.sigmoid(gates[:, :2 * D])
                g = jnp.tanh(gates[:, 2 * D:3 * D])
                o = jax.nn.sigmoid(gates[:, 3 * D:])
                c = i_f[:, D:] * c + i_f[:, :D] * g
                h = o * jnp.tanh(c)
                hs_s[pl.ds(t * B + off, HB), :] = h
                return h, c

            h1, c1 = half(h1, c1, 0)
            h2, c2 = half(h2, c2, HB)
        return h1, c1, h2, c2

    z = jnp.zeros((HB, D), jnp.float32)
    lax.fori_loop(0, S // UNROLL, step4, (z, z, z, z))

    ones = jnp.ones((D, 1), jnp.float32)
    logits = jnp.dot(hs_s[...] * qa_s[...], ones,
                     preferred_element_type=jnp.float32)
    out_ref[...] = jax.nn.sigmoid(logits)


def _k4(e0, n1, resp, w1, b1, wi, wh, bl, emb_r):
    return pl.pallas_call(
        _k4_body,
        out_shape=jax.ShapeDtypeStruct((SB, 1), jnp.float32),
        scratch_shapes=[
            pltpu.VMEM((SB, 4 * D), jnp.float32),
            pltpu.VMEM((SB, D), jnp.float32),
            pltpu.VMEM((SB, D), jnp.float32),
        ],
    )(e0, n1, resp, w1, b1, wi, wh, bl, emb_r)


def kernel(emb_q, emb_s, emb_r, W0, b0, W1, b1, Wi, Wh, bl,
           question, response, q_neighbors, s_neighbors):
    q_flat = jnp.transpose(question).reshape(-1).astype(jnp.int32)
    idx_q = q_flat.reshape(SB // TCHUNK, TCHUNK)

    idx_s = jnp.zeros((NS_PAD * NBS,), jnp.int32)
    idx_s = idx_s.at[:NS * NBS].set(
        s_neighbors.reshape(-1).astype(jnp.int32))
    idx_s = idx_s.reshape(NS_PAD * NBS // CHUNK, CHUNK)

    k1, k3 = _sc_kernels()
    m2, qnb = k1(emb_q, q_neighbors.astype(jnp.int32), idx_s, idx_q)

    a1_tab = _k2(m2, emb_s, W0, b0.reshape(1, D))

    idx3 = qnb.reshape(SB * NBQ // TCHUNK, TCHUNK)
    n1, e0 = k3(a1_tab, idx3, emb_q, idx_q)

    resp = jnp.transpose(response).reshape(SB, 1).astype(jnp.float32)
    pred_flat = _k4(e0, n1, resp, W1, b1.reshape(1, D), Wi, Wh,
                    bl.reshape(1, 4 * D), emb_r)
    return jnp.transpose(pred_flat.reshape(S, B))


# speedup vs baseline: 1.0079x; 1.0079x over previous
import functools

import jax
import jax.numpy as jnp
from jax import lax
from jax.experimental import pallas as pl
from jax.experimental.pallas import tpu as pltpu
from jax.experimental.pallas import tpu_sc as plsc

B, S = 32, 200
NQ, NS = 100000, 2000
NBQ, NBS = 8, 8
D = 64
SB = B * S
NW = 32
CHUNK = 128
TCHUNK = 100

NS_PAD = 2048
SK_PER_W = NS_PAD // NW
SK_ROWS = SK_PER_W * NBS
SK_CHUNKS = SK_ROWS // CHUNK

TOK_PER_W = SB // NW
TOK_CHUNKS = TOK_PER_W // TCHUNK

N1_CHUNKS = TOK_PER_W * NBQ // TCHUNK
N_SUPER = 4
CH_PER_SUPER = N1_CHUNKS // N_SUPER
SUPER_ROWS = CH_PER_SUPER * TCHUNK
TOK_PER_SUPER = SUPER_ROWS // NBQ


def _wid():
    return lax.axis_index("s") * 2 + lax.axis_index("c")


@functools.cache
def _sc_kernels():
    mesh = plsc.VectorSubcoreMesh(core_axis_name="c", subcore_axis_name="s")
    return _make_k1(mesh), _make_k3(mesh)


def _make_k1(mesh):
  return functools.partial(
    pl.kernel,
    out_type=(
        jax.ShapeDtypeStruct((NS_PAD, D), jnp.float32),
        jax.ShapeDtypeStruct((SB, NBQ), jnp.int32),
    ),
    mesh=mesh,
    compiler_params=pltpu.CompilerParams(use_tc_tiling_on_sc=False),
    scratch_types=[
        pltpu.VMEM((SK_CHUNKS, CHUNK), jnp.int32),
        pltpu.VMEM((TOK_CHUNKS, TCHUNK), jnp.int32),
        pltpu.VMEM((SK_ROWS, D), jnp.float32),
        pltpu.VMEM((SK_PER_W, D), jnp.float32),
        pltpu.VMEM((TOK_PER_W, NBQ), jnp.int32),
        pltpu.SemaphoreType.DMA,
        pltpu.SemaphoreType.DMA,
    ],
  )(_k1_body)


def _k1_body(emb_q_hbm, qnbrs_hbm, idx_s_hbm, idx_q_hbm,
             m2_out, qnb_out,
             idxs_v, idxq_v, rows_v, acc_v, qnb_v, sem0, sem1):
    wid = _wid()

    pltpu.sync_copy(idx_s_hbm.at[pl.ds(wid * SK_CHUNKS, SK_CHUNKS)], idxs_v)
    pltpu.sync_copy(idx_q_hbm.at[pl.ds(wid * TOK_CHUNKS, TOK_CHUNKS)], idxq_v)
    m2_cps = [
        pltpu.async_copy(emb_q_hbm.at[idxs_v.at[ch]],
                         rows_v.at[pl.ds(ch * CHUNK, CHUNK)], sem0)
        for ch in range(SK_CHUNKS)
    ]
    tok_cps = [
        pltpu.async_copy(qnbrs_hbm.at[idxq_v.at[ch]],
                         qnb_v.at[pl.ds(ch * TCHUNK, TCHUNK)], sem1)
        for ch in range(TOK_CHUNKS)
    ]
    for cp in m2_cps:
        cp.wait()

    def _acc_skill(sk, _):
        for c4 in range(D // 16):
            col = pl.ds(c4 * 16, 16)
            v = rows_v[sk * NBS, col]
            for j in range(1, NBS):
                v = v + rows_v[sk * NBS + j, col]
            acc_v[sk, col] = v * (1.0 / NBS)
        return 0

    lax.fori_loop(0, SK_PER_W, _acc_skill, 0)
    pltpu.sync_copy(acc_v, m2_out.at[pl.ds(wid * SK_PER_W, SK_PER_W)])

    for cp in tok_cps:
        cp.wait()
    pltpu.sync_copy(qnb_v, qnb_out.at[pl.ds(wid * TOK_PER_W, TOK_PER_W)])


def _make_k3(mesh):
  return functools.partial(
    pl.kernel,
    out_type=(
        jax.ShapeDtypeStruct((SB, D), jnp.float32),
        jax.ShapeDtypeStruct((SB, D), jnp.float32),
    ),
    mesh=mesh,
    compiler_params=pltpu.CompilerParams(use_tc_tiling_on_sc=False),
    scratch_types=[
        pltpu.VMEM((N1_CHUNKS, TCHUNK), jnp.int32),
        pltpu.VMEM((TOK_CHUNKS, TCHUNK), jnp.int32),
        pltpu.VMEM((SUPER_ROWS, D), jnp.float32),
        pltpu.VMEM((SUPER_ROWS, D), jnp.float32),
        pltpu.VMEM((TOK_PER_W, D), jnp.float32),
        pltpu.VMEM((TOK_PER_W, D), jnp.float32),
        pltpu.VMEM_SHARED((NS, D), jnp.float32),
        pltpu.SemaphoreType.DMA,
        pltpu.SemaphoreType.DMA,
        pltpu.SemaphoreType.DMA,
    ],
  )(_k3_body)


def _k3_body(a1_hbm, idx3_hbm, emb_q_hbm, idx_q_hbm,
             n1_out, e0_out,
             idx_v, idxq_v, rows0_v, rows1_v, acc_v, e0_v,
             a1_sp, sem0, sem1, sem2):
    wid = _wid()
    sid = lax.axis_index("s")

    pltpu.sync_copy(idx_q_hbm.at[pl.ds(wid * TOK_CHUNKS, TOK_CHUNKS)], idxq_v)
    e0_cps = [
        pltpu.async_copy(emb_q_hbm.at[idxq_v.at[ch]],
                         e0_v.at[pl.ds(ch * TCHUNK, TCHUNK)], sem2)
        for ch in range(TOK_CHUNKS)
    ]

    @pl.when(sid == 0)
    def _():
        pltpu.sync_copy(a1_hbm, a1_sp)

    pltpu.sync_copy(idx3_hbm.at[pl.ds(wid * N1_CHUNKS, N1_CHUNKS)], idx_v)
    plsc.subcore_barrier()
    bufs = [rows0_v, rows1_v]
    sems = [sem0, sem1]

    def fire(sc):
        buf, sem = bufs[sc % 2], sems[sc % 2]
        return [
            pltpu.async_copy(
                a1_sp.at[idx_v.at[sc * CH_PER_SUPER + ch]],
                buf.at[pl.ds(ch * TCHUNK, TCHUNK)], sem)
            for ch in range(CH_PER_SUPER)
        ]

    pending = fire(0)
    for sc in range(N_SUPER):
        for cp in pending:
            cp.wait()
        if sc + 1 < N_SUPER:
            pending = fire(sc + 1)
        buf = bufs[sc % 2]

        def _acc_tok(tl, _):
            for c4 in range(D // 16):
                col = pl.ds(c4 * 16, 16)
                v = buf[tl * NBQ, col]
                for j in range(1, NBQ):
                    v = v + buf[tl * NBQ + j, col]
                acc_v[sc * TOK_PER_SUPER + tl, col] = v * (1.0 / NBQ)
            return 0

        lax.fori_loop(0, TOK_PER_SUPER, _acc_tok, 0)

    pltpu.sync_copy(acc_v, n1_out.at[pl.ds(wid * TOK_PER_W, TOK_PER_W)])
    for cp in e0_cps:
        cp.wait()
    pltpu.sync_copy(e0_v, e0_out.at[pl.ds(wid * TOK_PER_W, TOK_PER_W)])


def _k2_body(m2_ref, emb_s_ref, w0_ref, b0_ref, out_ref):
    u = emb_s_ref[...] + m2_ref[:NS, :]
    out_ref[...] = jax.nn.relu(
        jnp.dot(u, w0_ref[...], preferred_element_type=jnp.float32)
        + b0_ref[...])


def _k2(m2, emb_s, w0, b0):
    return pl.pallas_call(
        _k2_body,
        out_shape=jax.ShapeDtypeStruct((NS, D), jnp.float32),
    )(m2, emb_s, w0, b0)


def _k4_body(e0_ref, n1_ref, resp_ref, w1_ref, b1_ref, wi_ref, wh_ref,
             bl_ref, embr_ref, out_ref, xw_s, qa_s, hs_s):
    e0 = e0_ref[:SB, :]
    n1 = n1_ref[:SB, :]
    q_agg = jax.nn.relu(
        jnp.dot(e0 + n1, w1_ref[...], preferred_element_type=jnp.float32)
        + b1_ref[...])
    qa_s[...] = q_agg

    wi_top = wi_ref[:D, :]
    wi_bot = wi_ref[D:, :]
    v0 = jnp.dot(embr_ref[0:1, :], wi_bot,
                 preferred_element_type=jnp.float32) + bl_ref[...]
    v1 = jnp.dot(embr_ref[1:2, :], wi_bot,
                 preferred_element_type=jnp.float32) + bl_ref[...]
    xw_s[...] = (jnp.dot(q_agg, wi_top, preferred_element_type=jnp.float32)
                 + v0 + resp_ref[...] * (v1 - v0))

    wh = wh_ref[...]

    UNROLL = 8

    def step4(t4, carry):
        h, c = carry
        for k in range(UNROLL):
            t = t4 * UNROLL + k
            xt = xw_s[pl.ds(t * B, B), :]
            gates = xt + jnp.dot(h, wh, preferred_element_type=jnp.float32)
            i_f = jax.nn.sigmoid(gates[:, :2 * D])
            g = jnp.tanh(gates[:, 2 * D:3 * D])
            o = jax.nn.sigmoid(gates[:, 3 * D:])
            c = i_f[:, D:] * c + i_f[:, :D] * g
            h = o * jnp.tanh(c)
            hs_s[pl.ds(t * B, B), :] = h
        return h, c

    h0 = jnp.zeros((B, D), jnp.float32)
    c0 = jnp.zeros((B, D), jnp.float32)
    lax.fori_loop(0, S // UNROLL, step4, (h0, c0))

    ones = jnp.ones((D, 1), jnp.float32)
    logits = jnp.dot(hs_s[...] * qa_s[...], ones,
                     preferred_element_type=jnp.float32)
    out_ref[...] = jax.nn.sigmoid(logits)


def _k4(e0, n1, resp, w1, b1, wi, wh, bl, emb_r):
    return pl.pallas_call(
        _k4_body,
        out_shape=jax.ShapeDtypeStruct((SB, 1), jnp.float32),
        scratch_shapes=[
            pltpu.VMEM((SB, 4 * D), jnp.float32),
            pltpu.VMEM((SB, D), jnp.float32),
            pltpu.VMEM((SB, D), jnp.float32),
        ],
    )(e0, n1, resp, w1, b1, wi, wh, bl, emb_r)


def kernel(emb_q, emb_s, emb_r, W0, b0, W1, b1, Wi, Wh, bl,
           question, response, q_neighbors, s_neighbors):
    q_flat = jnp.transpose(question).reshape(-1).astype(jnp.int32)
    idx_q = q_flat.reshape(SB // TCHUNK, TCHUNK)

    idx_s = jnp.zeros((NS_PAD * NBS,), jnp.int32)
    idx_s = idx_s.at[:NS * NBS].set(
        s_neighbors.reshape(-1).astype(jnp.int32))
    idx_s = idx_s.reshape(NS_PAD * NBS // CHUNK, CHUNK)

    k1, k3 = _sc_kernels()
    m2, qnb = k1(emb_q, q_neighbors.astype(jnp.int32), idx_s, idx_q)

    a1_tab = _k2(m2, emb_s, W0, b0.reshape(1, D))

    idx3 = qnb.reshape(SB * NBQ // TCHUNK, TCHUNK)
    n1, e0 = k3(a1_tab, idx3, emb_q, idx_q)

    resp = jnp.transpose(response).reshape(SB, 1).astype(jnp.float32)
    pred_flat = _k4(e0, n1, resp, W1, b1.reshape(1, D), Wi, Wh,
                    bl.reshape(1, 4 * D), emb_r)
    return jnp.transpose(pred_flat.reshape(S, B))


# speedup vs baseline: 1.0232x; 1.0152x over previous
import functools

import jax
import jax.numpy as jnp
from jax import lax
from jax.experimental import pallas as pl
from jax.experimental.pallas import tpu as pltpu
from jax.experimental.pallas import tpu_sc as plsc

B, S = 32, 200
NQ, NS = 100000, 2000
NBQ, NBS = 8, 8
D = 64
SB = B * S
NW = 32
CHUNK = 128
TCHUNK = 100

NS_PAD = 2048
SK_PER_W = NS_PAD // NW
SK_ROWS = SK_PER_W * NBS
SK_CHUNKS = SK_ROWS // CHUNK

TOK_PER_W = SB // NW
TOK_CHUNKS = TOK_PER_W // TCHUNK

N1_CHUNKS = TOK_PER_W * NBQ // TCHUNK
N_SUPER = 4
CH_PER_SUPER = N1_CHUNKS // N_SUPER
SUPER_ROWS = CH_PER_SUPER * TCHUNK
TOK_PER_SUPER = SUPER_ROWS // NBQ


def _wid():
    return lax.axis_index("s") * 2 + lax.axis_index("c")


@functools.cache
def _sc_kernels():
    mesh = plsc.VectorSubcoreMesh(core_axis_name="c", subcore_axis_name="s")
    return _make_k1(mesh), _make_k3(mesh)


def _make_k1(mesh):
  return functools.partial(
    pl.kernel,
    out_type=(
        jax.ShapeDtypeStruct((NS_PAD, D), jnp.float32),
        jax.ShapeDtypeStruct((SB, NBQ), jnp.int32),
    ),
    mesh=mesh,
    compiler_params=pltpu.CompilerParams(use_tc_tiling_on_sc=False),
    scratch_types=[
        pltpu.VMEM((SK_CHUNKS, CHUNK), jnp.int32),
        pltpu.VMEM((TOK_CHUNKS, TCHUNK), jnp.int32),
        pltpu.VMEM((SK_ROWS, D), jnp.float32),
        pltpu.VMEM((SK_PER_W, D), jnp.float32),
        pltpu.VMEM((TOK_PER_W, NBQ), jnp.int32),
        pltpu.SemaphoreType.DMA,
        pltpu.SemaphoreType.DMA,
    ],
  )(_k1_body)


def _k1_body(emb_q_hbm, qnbrs_hbm, idx_s_hbm, idx_q_hbm,
             m2_out, qnb_out,
             idxs_v, idxq_v, rows_v, acc_v, qnb_v, sem0, sem1):
    wid = _wid()

    pltpu.sync_copy(idx_s_hbm.at[pl.ds(wid * SK_CHUNKS, SK_CHUNKS)], idxs_v)
    pltpu.sync_copy(idx_q_hbm.at[pl.ds(wid * TOK_CHUNKS, TOK_CHUNKS)], idxq_v)
    m2_cps = [
        pltpu.async_copy(emb_q_hbm.at[idxs_v.at[ch]],
                         rows_v.at[pl.ds(ch * CHUNK, CHUNK)], sem0)
        for ch in range(SK_CHUNKS)
    ]
    tok_cps = [
        pltpu.async_copy(qnbrs_hbm.at[idxq_v.at[ch]],
                         qnb_v.at[pl.ds(ch * TCHUNK, TCHUNK)], sem1)
        for ch in range(TOK_CHUNKS)
    ]
    for cp in m2_cps:
        cp.wait()

    def _acc_skill(sk, _):
        for c4 in range(D // 16):
            col = pl.ds(c4 * 16, 16)
            v = rows_v[sk * NBS, col]
            for j in range(1, NBS):
                v = v + rows_v[sk * NBS + j, col]
            acc_v[sk, col] = v * (1.0 / NBS)
        return 0

    lax.fori_loop(0, SK_PER_W, _acc_skill, 0)
    pltpu.sync_copy(acc_v, m2_out.at[pl.ds(wid * SK_PER_W, SK_PER_W)])

    for cp in tok_cps:
        cp.wait()
    pltpu.sync_copy(qnb_v, qnb_out.at[pl.ds(wid * TOK_PER_W, TOK_PER_W)])


def _make_k3(mesh):
  return functools.partial(
    pl.kernel,
    out_type=(
        jax.ShapeDtypeStruct((SB, D), jnp.float32),
        jax.ShapeDtypeStruct((SB, D), jnp.float32),
    ),
    mesh=mesh,
    compiler_params=pltpu.CompilerParams(use_tc_tiling_on_sc=False),
    scratch_types=[
        pltpu.VMEM((N1_CHUNKS, TCHUNK), jnp.int32),
        pltpu.VMEM((TOK_CHUNKS, TCHUNK), jnp.int32),
        pltpu.VMEM((SUPER_ROWS, D), jnp.float32),
        pltpu.VMEM((SUPER_ROWS, D), jnp.float32),
        pltpu.VMEM((TOK_PER_W, D), jnp.float32),
        pltpu.VMEM((TOK_PER_W, D), jnp.float32),
        pltpu.VMEM_SHARED((NS, D), jnp.float32),
        pltpu.SemaphoreType.DMA,
        pltpu.SemaphoreType.DMA,
        pltpu.SemaphoreType.DMA,
    ],
  )(_k3_body)


def _k3_body(a1_hbm, idx3_hbm, emb_q_hbm, idx_q_hbm,
             n1_out, e0_out,
             idx_v, idxq_v, rows0_v, rows1_v, acc_v, e0_v,
             a1_sp, sem0, sem1, sem2):
    wid = _wid()
    sid = lax.axis_index("s")

    pltpu.sync_copy(idx_q_hbm.at[pl.ds(wid * TOK_CHUNKS, TOK_CHUNKS)], idxq_v)
    e0_cps = [
        pltpu.async_copy(emb_q_hbm.at[idxq_v.at[ch]],
                         e0_v.at[pl.ds(ch * TCHUNK, TCHUNK)], sem2)
        for ch in range(TOK_CHUNKS)
    ]

    @pl.when(sid == 0)
    def _():
        pltpu.sync_copy(a1_hbm, a1_sp)

    pltpu.sync_copy(idx3_hbm.at[pl.ds(wid * N1_CHUNKS, N1_CHUNKS)], idx_v)
    plsc.subcore_barrier()
    bufs = [rows0_v, rows1_v]
    sems = [sem0, sem1]

    def fire(sc):
        buf, sem = bufs[sc % 2], sems[sc % 2]
        return [
            pltpu.async_copy(
                a1_sp.at[idx_v.at[sc * CH_PER_SUPER + ch]],
                buf.at[pl.ds(ch * TCHUNK, TCHUNK)], sem)
            for ch in range(CH_PER_SUPER)
        ]

    pending = fire(0)
    for sc in range(N_SUPER):
        for cp in pending:
            cp.wait()
        if sc + 1 < N_SUPER:
            pending = fire(sc + 1)
        buf = bufs[sc % 2]

        def _acc_tok(tl, _):
            for c4 in range(D // 16):
                col = pl.ds(c4 * 16, 16)
                v = buf[tl * NBQ, col]
                for j in range(1, NBQ):
                    v = v + buf[tl * NBQ + j, col]
                acc_v[sc * TOK_PER_SUPER + tl, col] = v * (1.0 / NBQ)
            return 0

        lax.fori_loop(0, TOK_PER_SUPER, _acc_tok, 0)

    pltpu.sync_copy(acc_v, n1_out.at[pl.ds(wid * TOK_PER_W, TOK_PER_W)])
    for cp in e0_cps:
        cp.wait()
    pltpu.sync_copy(e0_v, e0_out.at[pl.ds(wid * TOK_PER_W, TOK_PER_W)])


def _k2_body(m2_ref, emb_s_ref, w0_ref, b0_ref, out_ref):
    u = emb_s_ref[...] + m2_ref[:NS, :]
    out_ref[...] = jax.nn.relu(
        jnp.dot(u, w0_ref[...], preferred_element_type=jnp.float32)
        + b0_ref[...])


def _k2(m2, emb_s, w0, b0):
    return pl.pallas_call(
        _k2_body,
        out_shape=jax.ShapeDtypeStruct((NS, D), jnp.float32),
    )(m2, emb_s, w0, b0)


def _k4_body(e0_ref, n1_ref, resp_ref, w1_ref, b1_ref, wi_ref, wh_ref,
             bl_ref, embr_ref, out_ref, xw_s, qa_s, hs_s):
    e0 = e0_ref[:SB, :]
    n1 = n1_ref[:SB, :]
    q_agg = jax.nn.relu(
        jnp.dot(e0 + n1, w1_ref[...], preferred_element_type=jnp.float32)
        + b1_ref[...])
    qa_s[...] = q_agg

    wi_top = wi_ref[:D, :]
    wi_bot = wi_ref[D:, :]
    v0 = jnp.dot(embr_ref[0:1, :], wi_bot,
                 preferred_element_type=jnp.float32) + bl_ref[...]
    v1 = jnp.dot(embr_ref[1:2, :], wi_bot,
                 preferred_element_type=jnp.float32) + bl_ref[...]
    xw_s[...] = (jnp.dot(q_agg, wi_top, preferred_element_type=jnp.float32)
                 + v0 + resp_ref[...] * (v1 - v0))

    wh = wh_ref[...]

    UNROLL = 8

    def step4(t4, carry):
        h, c = carry
        for k in range(UNROLL):
            t = t4 * UNROLL + k
            xt = xw_s[pl.ds(t * B, B), :]
            gates = xt + jnp.dot(h, wh, preferred_element_type=jnp.float32)
            i_f = jax.nn.sigmoid(gates[:, :2 * D])
            g = jnp.tanh(gates[:, 2 * D:3 * D])
            o = jax.nn.sigmoid(gates[:, 3 * D:])
            c = i_f[:, D:] * c + i_f[:, :D] * g
            h = o * jnp.tanh(c)
            hs_s[pl.ds(t * B, B), :] = h
        return h, c

    h0 = jnp.zeros((B, D), jnp.float32)
    c0 = jnp.zeros((B, D), jnp.float32)
    lax.fori_loop(0, S // UNROLL, step4, (h0, c0))

    ones = jnp.ones((1, D), jnp.float32)
    logits = lax.dot_general(ones, hs_s[...] * qa_s[...],
                             (((1,), (1,)), ((), ())),
                             preferred_element_type=jnp.float32)
    out_ref[...] = jax.nn.sigmoid(logits)


def _k4(e0, n1, resp, w1, b1, wi, wh, bl, emb_r):
    return pl.pallas_call(
        _k4_body,
        out_shape=jax.ShapeDtypeStruct((1, SB), jnp.float32),
        scratch_shapes=[
            pltpu.VMEM((SB, 4 * D), jnp.float32),
            pltpu.VMEM((SB, D), jnp.float32),
            pltpu.VMEM((SB, D), jnp.float32),
        ],
    )(e0, n1, resp, w1, b1, wi, wh, bl, emb_r)


def kernel(emb_q, emb_s, emb_r, W0, b0, W1, b1, Wi, Wh, bl,
           question, response, q_neighbors, s_neighbors):
    q_flat = jnp.transpose(question).reshape(-1).astype(jnp.int32)
    idx_q = q_flat.reshape(SB // TCHUNK, TCHUNK)

    idx_s = jnp.zeros((NS_PAD * NBS,), jnp.int32)
    idx_s = idx_s.at[:NS * NBS].set(
        s_neighbors.reshape(-1).astype(jnp.int32))
    idx_s = idx_s.reshape(NS_PAD * NBS // CHUNK, CHUNK)

    k1, k3 = _sc_kernels()
    m2, qnb = k1(emb_q, q_neighbors.astype(jnp.int32), idx_s, idx_q)

    a1_tab = _k2(m2, emb_s, W0, b0.reshape(1, D))

    idx3 = qnb.reshape(SB * NBQ // TCHUNK, TCHUNK)
    n1, e0 = k3(a1_tab, idx3, emb_q, idx_q)

    resp = jnp.transpose(response).reshape(SB, 1).astype(jnp.float32)
    pred_flat = _k4(e0, n1, resp, W1, b1.reshape(1, D), Wi, Wh,
                    bl.reshape(1, 4 * D), emb_r)
    return jnp.transpose(pred_flat.reshape(S, B))
